# baseline (speedup 1.0000x reference)
import functools

import jax
import jax.numpy as jnp
from jax import lax
from jax.experimental import pallas as pl
from jax.experimental.pallas import tpu as pltpu
from jax.experimental.pallas import tpu_sc as plsc

N = 10000
E = 160000
D = 128
DE = 42
KA = 16
KC = 6
KB = 5
RK = 16
DEPTH = 2
F = 128
HP = 64
XW = 16
DXW = 16

NC = 2
NS = 16
NW = NC * NS
CH = 128
EH = E // 2
NCHUNK = EH // CH
CPW = -(-NCHUNK // NW)
H0 = -(-NCHUNK // NC)
SPW = -(-H0 // NS)
RPS = N // NS
NP = 10112
RPSP = NP // NS

BE = 4000
BN = 2000

_f32 = jnp.float32
_bf16 = jnp.bfloat16


@functools.lru_cache(maxsize=None)
def _sc_kernels():
    mesh = plsc.VectorSubcoreMesh(core_axis_name="c", subcore_axis_name="s")

    @functools.partial(
        pl.kernel,
        mesh=mesh,
        out_type=(jax.ShapeDtypeStruct((EH, F), _f32),
                  jax.ShapeDtypeStruct((EH, F), _f32)),
        scratch_types=[
            pltpu.VMEM((CH,), jnp.int32),
            pltpu.VMEM((CH,), jnp.int32),
            pltpu.VMEM((CH,), jnp.int32),
            pltpu.VMEM((CH,), jnp.int32),
            pltpu.VMEM((CH, F), _f32),
            pltpu.VMEM((CH, F), _f32),
            pltpu.VMEM((CH, F), _f32),
            pltpu.VMEM((CH, F), _f32),
            pltpu.SemaphoreType.DMA,
            pltpu.SemaphoreType.DMA,
            pltpu.SemaphoreType.DMA,
            pltpu.SemaphoreType.DMA,
            pltpu.SemaphoreType.DMA,
            pltpu.SemaphoreType.DMA,
        ],
    )
    def gather_kernel(feat_hbm, src_hbm, dst_hbm, os_hbm, od_hbm,
                      sidx0, sidx1, didx0, didx1, srow0, srow1, drow0, drow1,
                      isem0, isem1, gsem0, gsem1, wsem0, wsem1):
        wid = lax.axis_index("s") * NC + lax.axis_index("c")
        sidx = (sidx0, sidx1)
        didx = (didx0, didx1)
        srow = (srow0, srow1)
        drow = (drow0, drow1)
        isem = (isem0, isem1)
        gsem = (gsem0, gsem1)
        wsem = (wsem0, wsem1)

        def issue_idx(b, k):
            @pl.when(wid + k * NW < NCHUNK)
            def _():
                base = (wid + k * NW) * CH
                pltpu.async_copy(src_hbm.at[pl.ds(base, CH)], sidx[b], isem[b])
                pltpu.async_copy(dst_hbm.at[pl.ds(base, CH)], didx[b], isem[b])

        issue_idx(0, 0)

        @pl.loop(0, CPW, step=2)
        def _(k):
            for b in (0, 1):
                kk = k + b

                @pl.when(wid + kk * NW < NCHUNK)
                def _():
                    base = (wid + kk * NW) * CH

                    @pl.when(kk >= 2)
                    def _():
                        pltpu.make_async_copy(
                            srow[b], os_hbm.at[pl.ds(base, CH)], wsem[b]).wait()
                        pltpu.make_async_copy(
                            drow[b], od_hbm.at[pl.ds(base, CH)], wsem[b]).wait()
                    pltpu.make_async_copy(
                        src_hbm.at[pl.ds(base, CH)], sidx[b], isem[b]).wait()
                    pltpu.make_async_copy(
                        dst_hbm.at[pl.ds(base, CH)], didx[b], isem[b]).wait()
                    pltpu.async_copy(feat_hbm.at[sidx[b]], srow[b], gsem[b])
                    pltpu.async_copy(feat_hbm.at[didx[b]], drow[b], gsem[b])
                    issue_idx(1 - b, kk + 1)
                    pltpu.make_async_copy(
                        feat_hbm.at[sidx[b]], srow[b], gsem[b]).wait()
                    pltpu.make_async_copy(
                        feat_hbm.at[didx[b]], drow[b], gsem[b]).wait()
                    pltpu.async_copy(srow[b], os_hbm.at[pl.ds(base, CH)],
                                     wsem[b])
                    pltpu.async_copy(drow[b], od_hbm.at[pl.ds(base, CH)],
                                     wsem[b])

        nw_c = (NCHUNK - wid + NW - 1) // NW
        for b in (0, 1):
            @pl.when(nw_c > b)
            def _():
                pltpu.make_async_copy(
                    srow[b], os_hbm.at[pl.ds(0, CH)], wsem[b]).wait()
                pltpu.make_async_copy(
                    drow[b], od_hbm.at[pl.ds(0, CH)], wsem[b]).wait()

    @functools.partial(
        pl.kernel,
        mesh=mesh,
        out_type=jax.ShapeDtypeStruct((NC, NP, D), _f32),
        scratch_types=[
            pltpu.VMEM((CH,), jnp.int32),
            pltpu.VMEM((CH,), jnp.int32),
            pltpu.VMEM((CH, D), _f32),
            pltpu.VMEM((CH, D), _f32),
            pltpu.VMEM_SHARED((NP, D), _f32),
            pltpu.SemaphoreType.DMA,
            pltpu.SemaphoreType.DMA,
        ],
    )
    def scatter_pm_kernel(pm_hbm, dst_hbm, zeros_hbm, out_hbm,
                          idxv0, idxv1, rowv0, rowv1, shared,
                          dsem0, dsem1):
        cid = lax.axis_index("c")
        sid = lax.axis_index("s")
        r0 = sid * RPSP
        idxv = (idxv0, idxv1)
        rowv = (rowv0, rowv1)
        dsem = (dsem0, dsem1)
        lim = H0 - cid

        def issue(b, k):
            @pl.when(sid + k * NS < lim)
            def _():
                base = (cid * H0 + sid + k * NS) * CH
                pltpu.async_copy(dst_hbm.at[pl.ds(base, CH)], idxv[b], dsem[b])
                pltpu.async_copy(pm_hbm.at[pl.ds(base, CH)], rowv[b], dsem[b])

        issue(0, 0)
        pltpu.sync_copy(zeros_hbm.at[pl.ds(r0, RPSP)],
                        shared.at[pl.ds(r0, RPSP)])
        plsc.subcore_barrier()

        @pl.loop(0, SPW, step=2)
        def _(k):
            for b in (0, 1):
                kk = k + b

                @pl.when(sid + kk * NS < lim)
                def _():
                    pltpu.make_async_copy(
                        dst_hbm.at[pl.ds(0, CH)], idxv[b], dsem[b]).wait()
                    pltpu.make_async_copy(
                        pm_hbm.at[pl.ds(0, CH)], rowv[b], dsem[b]).wait()
                    issue(1 - b, kk + 1)
                    pltpu.sync_copy(rowv[b], shared.at[idxv[b]], add=True)

        plsc.subcore_barrier()
        pltpu.sync_copy(shared.at[pl.ds(r0, RPSP)],
                        out_hbm.at[cid, pl.ds(r0, RPSP)])

    @functools.partial(
        pl.kernel,
        mesh=mesh,
        compiler_params=pltpu.CompilerParams(use_tc_tiling_on_sc=False),
        out_type=jax.ShapeDtypeStruct((NC, N, DXW), _f32),
        scratch_types=[
            pltpu.VMEM((CH,), jnp.int32),
            pltpu.VMEM((CH, DXW), _f32),
            pltpu.VMEM_SHARED((N, DXW), _f32),
            pltpu.SemaphoreType.DMA,
        ],
    )
    def scatter_dx_kernel(dx_hbm, dst_hbm, zeros_hbm, out_hbm,
                          idxv, rowv, shared, sem):
        cid = lax.axis_index("c")
        sid = lax.axis_index("s")
        r0 = sid * RPS
        pltpu.sync_copy(zeros_hbm.at[pl.ds(r0, RPS)],
                        shared.at[pl.ds(r0, RPS)])
        plsc.subcore_barrier()

        @pl.loop(0, SPW)
        def _(i):
            local = sid + i * NS

            @pl.when(local < H0 - cid)
            def _():
                base = (cid * H0 + local) * CH
                pltpu.sync_copy(dst_hbm.at[pl.ds(base, CH)], idxv)
                pltpu.sync_copy(dx_hbm.at[pl.ds(base, CH)], rowv)
                pltpu.sync_copy(rowv, shared.at[idxv], add=True)

        plsc.subcore_barrier()
        pltpu.sync_copy(shared.at[pl.ds(r0, RPS)],
                        out_hbm.at[cid, pl.ds(r0, RPS)])

    return gather_kernel, scatter_pm_kernel, scatter_dx_kernel


def _gather_pair(feat, src, dst):
    return _sc_kernels()[0](feat, src, dst)


def _segment_pm(pm, dst, zeros_pm):
    return _sc_kernels()[1](pm, dst, zeros_pm)


def _segment_dx(dx, dst, zeros_dx):
    return _sc_kernels()[2](dx, dst, zeros_dx)


def _init_body(ia_ref, ic_ref, xp_ref, A_ref, C_ref, fb_ref, feat_ref):
    ia = ia_ref[0, 0, :]
    ic = ic_ref[0, 0, :]
    oa = (ia[:, None] == lax.broadcasted_iota(jnp.int32, (1, KA), 1)).astype(_f32)
    oc = (ic[:, None] == lax.broadcasted_iota(jnp.int32, (1, 8), 1)).astype(_f32)
    h0 = (jnp.dot(oa, A_ref[...], preferred_element_type=_f32)
          + jnp.dot(oc, C_ref[...], preferred_element_type=_f32)
          + fb_ref[...])
    feat_ref[:, :HP] = _pack_h(h0)
    feat_ref[:, HP:HP + XW] = xp_ref[...]
    feat_ref[:, HP + XW:] = jnp.zeros((feat_ref.shape[0], F - HP - XW), _f32)


def _init_feat(ia3, ic3, xp, A_tab, C_tab8, fuse_b_row):
    nb = N // BN
    return pl.pallas_call(
        _init_body,
        grid=(nb,),
        in_specs=[
            pl.BlockSpec((1, 1, BN), lambda i: (i, 0, 0)),
            pl.BlockSpec((1, 1, BN), lambda i: (i, 0, 0)),
            pl.BlockSpec((BN, XW), lambda i: (i, 0)),
            pl.BlockSpec((KA, D), lambda i: (0, 0)),
            pl.BlockSpec((8, D), lambda i: (0, 0)),
            pl.BlockSpec((1, D), lambda i: (0, 0)),
        ],
        out_specs=pl.BlockSpec((BN, F), lambda i: (i, 0)),
        out_shape=jax.ShapeDtypeStruct((N, F), _f32),
    )(ia3, ic3, xp, A_tab, C_tab8, fuse_b_row)


_CEN_STEP = 10.0 / (RK - 1)
_INV_W = 1.0 / (10.0 / RK + 1e-12)


def _silu(v):
    return v * jax.nn.sigmoid(v)


def _pack_h(h):
    lo = lax.bitcast_convert_type(h[:, :HP].astype(_bf16).astype(_f32),
                                  jnp.uint32)
    hi = lax.bitcast_convert_type(h[:, HP:].astype(_bf16).astype(_f32),
                                  jnp.uint32)
    w = (lo >> 16) | (hi & jnp.uint32(0xFFFF0000))
    return lax.bitcast_convert_type(w, _f32)


def _unpack_h(pk):
    w = lax.bitcast_convert_type(pk, jnp.uint32)
    lo = lax.bitcast_convert_type(w << 16, _f32)
    hi = lax.bitcast_convert_type(w & jnp.uint32(0xFFFF0000), _f32)
    return jnp.concatenate([lo, hi], axis=1)


def _unpack_h_bf(pk):
    w = lax.bitcast_convert_type(pk, jnp.uint32)
    lo = lax.bitcast_convert_type(w << 16, _f32).astype(_bf16)
    hi = lax.bitcast_convert_type(w & jnp.uint32(0xFFFF0000), _f32).astype(_bf16)
    return lo, hi


def _silu_bf(v):
    vb = v.astype(_bf16)
    return vb * jax.nn.sigmoid(vb)


def _edge_body(hs_ref, hd_ref, ib_ref, W1d_ref, W1s_ref, W1r_ref, Et_ref,
               b1_ref, W2m_ref, b2m_ref, W2p_ref, b2p_ref, Wx1_ref, bx1_ref,
               wx2_ref, bx2_ref, pm_ref, dx_ref):
    hs_lo, hs_hi = _unpack_h_bf(hs_ref[:, :HP])
    hd_lo, hd_hi = _unpack_h_bf(hd_ref[:, :HP])
    xs = hs_ref[:, HP:HP + XW]
    xd = hd_ref[:, HP:HP + XW]
    rij = xd - xs
    dsq = jnp.sum(rij * rij, axis=1, keepdims=True)
    dist = jnp.sqrt(dsq)
    cen = lax.broadcasted_iota(jnp.int32, (1, RK), 1).astype(_f32) * _CEN_STEP
    z = (dist - cen) * _INV_W
    rbf = jnp.exp(-0.5 * z * z)
    ib = ib_ref[0, 0, :]
    oh = (ib[:, None] == lax.broadcasted_iota(jnp.int32, (1, 8), 1)).astype(_bf16)

    pre = (jnp.dot(hd_lo, W1d_ref[:HP], preferred_element_type=_f32)
           + jnp.dot(hd_hi, W1d_ref[HP:], preferred_element_type=_f32)
           + jnp.dot(hs_lo, W1s_ref[:HP], preferred_element_type=_f32)
           + jnp.dot(hs_hi, W1s_ref[HP:], preferred_element_type=_f32)
           + jnp.dot(rbf.astype(_bf16), W1r_ref[...], preferred_element_type=_f32)
           + jnp.dot(oh, Et_ref[...], preferred_element_type=_f32))
    m = _silu(pre[:, :D])
    pp = _silu(pre[:, D:])
    m2 = _silu(jnp.dot(m.astype(_bf16), W2m_ref[...],
                       preferred_element_type=_f32) + b2m_ref[...])
    pm = _silu(jnp.dot(pp.astype(_bf16), W2p_ref[...],
                       preferred_element_type=_f32) + b2p_ref[...])
    g1 = _silu(jnp.dot(m2.astype(_bf16), Wx1_ref[...],
                       preferred_element_type=_f32) + bx1_ref[...])
    gate = jnp.sum(g1 * wx2_ref[...], axis=1, keepdims=True) + bx2_ref[...]
    pm_ref[...] = pm
    dx_ref[...] = rij * gate


def _edge_call(hs, hd, ib3, w):
    nb = EH // BE
    full = lambda a: pl.BlockSpec(a.shape, lambda i: (0,) * a.ndim)
    return pl.pallas_call(
        _edge_body,
        grid=(nb,),
        in_specs=[
            pl.BlockSpec((BE, F), lambda i: (i, 0)),
            pl.BlockSpec((BE, F), lambda i: (i, 0)),
            pl.BlockSpec((1, 1, BE), lambda i: (i, 0, 0)),
        ] + [full(a) for a in w],
        out_specs=[
            pl.BlockSpec((BE, D), lambda i: (i, 0)),
            pl.BlockSpec((BE, DXW), lambda i: (i, 0)),
        ],
        out_shape=[
            jax.ShapeDtypeStruct((EH, D), _f32),
            jax.ShapeDtypeStruct((EH, DXW), _f32),
        ],
    )(hs, hd, ib3, *w)


def _node_body(feat_ref, p0_ref, p1_ref, p2_ref, p3_ref,
               q0_ref, q1_ref, q2_ref, q3_ref, Wh_ref, Wp_ref,
               b1_ref, W2_ref, b2_ref, g_ref, bln_ref, nf_ref, hsum_ref):
    i = pl.program_id(0)
    h = _unpack_h(feat_ref[:, :HP])
    xsl = feat_ref[:, HP:HP + XW]
    pm = (p0_ref[0] + p1_ref[0]) + (p2_ref[0] + p3_ref[0])
    dx = (q0_ref[0] + q1_ref[0]) + (q2_ref[0] + q3_ref[0])
    u1 = _silu(jnp.dot(h.astype(_bf16), Wh_ref[...], preferred_element_type=_f32)
               + jnp.dot(pm.astype(_bf16), Wp_ref[...], preferred_element_type=_f32)
               + b1_ref[...])
    u = jnp.dot(u1.astype(_bf16), W2_ref[...],
                preferred_element_type=_f32) + b2_ref[...]
    hr = h + u
    mu = jnp.mean(hr, axis=1, keepdims=True)
    var = jnp.mean((hr - mu) ** 2, axis=1, keepdims=True)
    hn = (hr - mu) / jnp.sqrt(var + 1e-5) * g_ref[...] + bln_ref[...]
    nf_ref[:, :HP] = _pack_h(hn)
    nf_ref[:, HP:HP + XW] = xsl + dx
    nf_ref[:, HP + XW:] = jnp.zeros((nf_ref.shape[0], F - HP - XW), _f32)
    ps = jnp.sum(hn, axis=0, keepdims=True)

    @pl.when(i == 0)
    def _():
        hsum_ref[...] = ps

    @pl.when(i > 0)
    def _():
        hsum_ref[...] += ps


def _node_call(feat, pm_a, pm_b, dx_a, dx_b, w):
    nb = N // BN
    full = lambda a: pl.BlockSpec(a.shape, lambda i: (0,) * a.ndim)
    return pl.pallas_call(
        _node_body,
        grid=(nb,),
        in_specs=[
            pl.BlockSpec((BN, F), lambda i: (i, 0)),
            pl.BlockSpec((1, BN, D), lambda i: (0, i, 0)),
            pl.BlockSpec((1, BN, D), lambda i: (1, i, 0)),
            pl.BlockSpec((1, BN, D), lambda i: (0, i, 0)),
            pl.BlockSpec((1, BN, D), lambda i: (1, i, 0)),
            pl.BlockSpec((1, BN, DXW), lambda i: (0, i, 0)),
            pl.BlockSpec((1, BN, DXW), lambda i: (1, i, 0)),
            pl.BlockSpec((1, BN, DXW), lambda i: (0, i, 0)),
            pl.BlockSpec((1, BN, DXW), lambda i: (1, i, 0)),
        ] + [full(a) for a in w],
        out_specs=[
            pl.BlockSpec((BN, F), lambda i: (i, 0)),
            pl.BlockSpec((1, D), lambda i: (0, 0)),
        ],
        out_shape=[
            jax.ShapeDtypeStruct((N, F), _f32),
            jax.ShapeDtypeStruct((1, D), _f32),
        ],
    )(feat, pm_a, pm_a, pm_b, pm_b, dx_a, dx_a, dx_b, dx_b, *w)


def kernel(a_t, c_t, x_t, e_t, edge_index, W_atom, W_charge, W_bond, fuse_W,
           fuse_b, lift_W, lift_b, phim_W1, phim_b1, phim_W2, phim_b2,
           phix_W1, phix_b1, phix_W2, phix_b2, psim_W1, psim_b1, psim_W2,
           psim_b2, upd_W1, upd_b1, upd_W2, upd_b2, ln_g, ln_b, head_W,
           head_b):
    def amax(probs, key):
        logits = jnp.log(jnp.maximum(probs, 1e-12))
        g = jax.random.gumbel(key, logits.shape, logits.dtype)
        return jnp.argmax(logits + g, axis=-1).astype(jnp.int32)

    ia = amax(a_t, jax.random.key(11))
    ic = amax(c_t, jax.random.key(12))
    ib = amax(e_t, jax.random.key(13))
    ia3 = ia.reshape(N // BN, 1, BN)
    ic3 = ic.reshape(N // BN, 1, BN)
    ib3a = ib[:EH].reshape(EH // BE, 1, BE)
    ib3b = ib[EH:].reshape(EH // BE, 1, BE)

    A_tab = W_atom @ fuse_W[:DE]
    C_tab = W_charge @ fuse_W[DE:2 * DE]
    C_tab8 = jnp.zeros((8, D), _f32).at[:KC].set(C_tab)
    B_tab = W_bond @ lift_W
    fuse_b_row = fuse_b.reshape(1, D)

    xp = jnp.concatenate([x_t, jnp.zeros((N, XW - 3), _f32)], axis=1)
    feat = _init_feat(ia3, ic3, xp, A_tab, C_tab8, fuse_b_row)

    src_a = edge_index[0, :EH]
    src_b = edge_index[0, EH:]
    dst_a = edge_index[1, :EH]
    dst_b = edge_index[1, EH:]
    zeros_pm = jnp.zeros((NP, D), _f32)
    zeros_dx = jnp.zeros((N, DXW), _f32)

    edge_w = []
    node_w = []
    for l in range(DEPTH):
        W1 = jnp.concatenate([phim_W1[l], psim_W1[l]], axis=1)
        b1v = (jnp.concatenate([phim_b1[l], psim_b1[l]])
               + lift_b @ W1[2 * D + RK:])
        Et = jnp.zeros((8, 2 * D), _f32).at[:KB].set(B_tab @ W1[2 * D + RK:]
                                                     + b1v)
        b1 = jnp.zeros((1, 2 * D), _f32)
        edge_w.append([
            W1[:D].astype(_bf16),
            W1[D:2 * D].astype(_bf16),
            W1[2 * D:2 * D + RK].astype(_bf16),
            Et.astype(_bf16),
            b1,
            phim_W2[l].astype(_bf16),
            phim_b2[l].reshape(1, D),
            psim_W2[l].astype(_bf16),
            psim_b2[l].reshape(1, D),
            phix_W1[l].astype(_bf16),
            phix_b1[l].reshape(1, D),
            phix_W2[l].reshape(1, D),
            phix_b2[l].reshape(1, 1),
        ])
        node_w.append([
            upd_W1[l][:D].astype(_bf16),
            upd_W1[l][D:].astype(_bf16),
            upd_b1[l].reshape(1, D),
            upd_W2[l].astype(_bf16),
            upd_b2[l].reshape(1, D),
            ln_g[l].reshape(1, D),
            ln_b[l].reshape(1, D),
        ])

    hsum = None
    for l in range(DEPTH):
        hs_a, hd_a = _gather_pair(feat, src_a, dst_a)
        hs_b, hd_b = _gather_pair(feat, src_b, dst_b)
        pm_a, dx_a = _edge_call(hs_a, hd_a, ib3a, edge_w[l])
        pm_b, dx_b = _edge_call(hs_b, hd_b, ib3b, edge_w[l])
        parts_pm_a = _segment_pm(pm_a, dst_a, zeros_pm)
        parts_pm_b = _segment_pm(pm_b, dst_b, zeros_pm)
        parts_dx_a = _segment_dx(dx_a, dst_a, zeros_dx)
        parts_dx_b = _segment_dx(dx_b, dst_b, zeros_dx)
        feat, hsum = _node_call(feat, parts_pm_a, parts_pm_b,
                                parts_dx_a, parts_dx_b, node_w[l])

    hg = hsum[0] / N
    out = hg @ head_W + head_b
    return jax.nn.sigmoid(out / 2.0)


# speedup vs baseline: 1.1165x; 1.1165x over previous
import functools

import jax
import jax.numpy as jnp
from jax import lax
from jax.experimental import pallas as pl
from jax.experimental.pallas import tpu as pltpu
from jax.experimental.pallas import tpu_sc as plsc

N = 10000
E = 160000
D = 128
DE = 42
KA = 16
KC = 6
KB = 5
RK = 16
DEPTH = 2
F = 128
HP = 64
XW = 16
DXW = 16

NC = 2
NS = 16
NW = NC * NS
CH = 128
EH = E // 2
NCHUNK = EH // CH
CPW = -(-NCHUNK // NW)
H0 = -(-NCHUNK // NC)
SPW = -(-H0 // NS)
RPS = N // NS
NP = 10112
RPSP = NP // NS

BE = 4000
BN = 2000

_f32 = jnp.float32
_bf16 = jnp.bfloat16


@functools.lru_cache(maxsize=None)
def _sc_kernels():
    mesh = plsc.VectorSubcoreMesh(core_axis_name="c", subcore_axis_name="s")

    @functools.partial(
        pl.kernel,
        mesh=mesh,
        out_type=(jax.ShapeDtypeStruct((EH, F), _f32),
                  jax.ShapeDtypeStruct((EH, F), _f32)),
        scratch_types=[
            pltpu.VMEM((CH,), jnp.int32),
            pltpu.VMEM((CH,), jnp.int32),
            pltpu.VMEM((CH,), jnp.int32),
            pltpu.VMEM((CH,), jnp.int32),
            pltpu.VMEM((CH, F), _f32),
            pltpu.VMEM((CH, F), _f32),
            pltpu.VMEM((CH, F), _f32),
            pltpu.VMEM((CH, F), _f32),
            pltpu.SemaphoreType.DMA,
            pltpu.SemaphoreType.DMA,
            pltpu.SemaphoreType.DMA,
            pltpu.SemaphoreType.DMA,
            pltpu.SemaphoreType.DMA,
            pltpu.SemaphoreType.DMA,
        ],
    )
    def gather_kernel(feat_hbm, src_hbm, dst_hbm, os_hbm, od_hbm,
                      sidx0, sidx1, didx0, didx1, srow0, srow1, drow0, drow1,
                      isem0, isem1, gsem0, gsem1, wsem0, wsem1):
        wid = lax.axis_index("s") * NC + lax.axis_index("c")
        sidx = (sidx0, sidx1)
        didx = (didx0, didx1)
        srow = (srow0, srow1)
        drow = (drow0, drow1)
        isem = (isem0, isem1)
        gsem = (gsem0, gsem1)
        wsem = (wsem0, wsem1)

        def issue_idx(b, k):
            @pl.when(wid + k * NW < NCHUNK)
            def _():
                base = (wid + k * NW) * CH
                pltpu.async_copy(src_hbm.at[pl.ds(base, CH)], sidx[b], isem[b])
                pltpu.async_copy(dst_hbm.at[pl.ds(base, CH)], didx[b], isem[b])

        issue_idx(0, 0)

        @pl.loop(0, CPW, step=2)
        def _(k):
            for b in (0, 1):
                kk = k + b

                @pl.when(wid + kk * NW < NCHUNK)
                def _():
                    base = (wid + kk * NW) * CH

                    @pl.when(kk >= 2)
                    def _():
                        pltpu.make_async_copy(
                            srow[b], os_hbm.at[pl.ds(base, CH)], wsem[b]).wait()
                        pltpu.make_async_copy(
                            drow[b], od_hbm.at[pl.ds(base, CH)], wsem[b]).wait()
                    pltpu.make_async_copy(
                        src_hbm.at[pl.ds(base, CH)], sidx[b], isem[b]).wait()
                    pltpu.make_async_copy(
                        dst_hbm.at[pl.ds(base, CH)], didx[b], isem[b]).wait()
                    pltpu.async_copy(feat_hbm.at[sidx[b]], srow[b], gsem[b])
                    pltpu.async_copy(feat_hbm.at[didx[b]], drow[b], gsem[b])
                    issue_idx(1 - b, kk + 1)
                    pltpu.make_async_copy(
                        feat_hbm.at[sidx[b]], srow[b], gsem[b]).wait()
                    pltpu.make_async_copy(
                        feat_hbm.at[didx[b]], drow[b], gsem[b]).wait()
                    pltpu.async_copy(srow[b], os_hbm.at[pl.ds(base, CH)],
                                     wsem[b])
                    pltpu.async_copy(drow[b], od_hbm.at[pl.ds(base, CH)],
                                     wsem[b])

        nw_c = (NCHUNK - wid + NW - 1) // NW
        for b in (0, 1):
            @pl.when(nw_c > b)
            def _():
                pltpu.make_async_copy(
                    srow[b], os_hbm.at[pl.ds(0, CH)], wsem[b]).wait()
                pltpu.make_async_copy(
                    drow[b], od_hbm.at[pl.ds(0, CH)], wsem[b]).wait()

    @functools.partial(
        pl.kernel,
        mesh=mesh,
        out_type=jax.ShapeDtypeStruct((NC, NP, D), _f32),
        scratch_types=[
            pltpu.VMEM((CH,), jnp.int32),
            pltpu.VMEM((CH,), jnp.int32),
            pltpu.VMEM((CH, D), _f32),
            pltpu.VMEM((CH, D), _f32),
            pltpu.VMEM_SHARED((NP, D), _f32),
            pltpu.SemaphoreType.DMA,
            pltpu.SemaphoreType.DMA,
        ],
    )
    def scatter_pm_kernel(pm_hbm, dst_hbm, zeros_hbm, out_hbm,
                          idxv0, idxv1, rowv0, rowv1, shared,
                          dsem0, dsem1):
        cid = lax.axis_index("c")
        sid = lax.axis_index("s")
        r0 = sid * RPSP
        idxv = (idxv0, idxv1)
        rowv = (rowv0, rowv1)
        dsem = (dsem0, dsem1)
        lim = H0 - cid

        def issue(b, k):
            @pl.when(sid + k * NS < lim)
            def _():
                base = (cid * H0 + sid + k * NS) * CH
                pltpu.async_copy(dst_hbm.at[pl.ds(base, CH)], idxv[b], dsem[b])
                pltpu.async_copy(pm_hbm.at[pl.ds(base, CH)], rowv[b], dsem[b])

        issue(0, 0)
        pltpu.sync_copy(zeros_hbm.at[pl.ds(r0, RPSP)],
                        shared.at[pl.ds(r0, RPSP)])
        plsc.subcore_barrier()

        @pl.loop(0, SPW, step=2)
        def _(k):
            for b in (0, 1):
                kk = k + b

                @pl.when(sid + kk * NS < lim)
                def _():
                    pltpu.make_async_copy(
                        dst_hbm.at[pl.ds(0, CH)], idxv[b], dsem[b]).wait()
                    pltpu.make_async_copy(
                        pm_hbm.at[pl.ds(0, CH)], rowv[b], dsem[b]).wait()
                    issue(1 - b, kk + 1)
                    pltpu.sync_copy(rowv[b], shared.at[idxv[b]], add=True)

        plsc.subcore_barrier()
        pltpu.sync_copy(shared.at[pl.ds(r0, RPSP)],
                        out_hbm.at[cid, pl.ds(r0, RPSP)])

    @functools.partial(
        pl.kernel,
        mesh=mesh,
        out_type=jax.ShapeDtypeStruct((NC, NP, DXW), _f32),
        scratch_types=[
            pltpu.VMEM((CH,), jnp.int32),
            pltpu.VMEM((CH, DXW), _f32),
            pltpu.VMEM_SHARED((NP, DXW), _f32),
            pltpu.SemaphoreType.DMA,
        ],
    )
    def scatter_dx_kernel(dx_hbm, dst_hbm, zeros_hbm, out_hbm,
                          idxv, rowv, shared, sem):
        cid = lax.axis_index("c")
        sid = lax.axis_index("s")
        r0 = sid * RPSP
        pltpu.sync_copy(zeros_hbm.at[pl.ds(r0, RPSP)],
                        shared.at[pl.ds(r0, RPSP)])
        plsc.subcore_barrier()

        @pl.loop(0, SPW)
        def _(i):
            local = sid + i * NS

            @pl.when(local < H0 - cid)
            def _():
                base = (cid * H0 + local) * CH
                pltpu.sync_copy(dst_hbm.at[pl.ds(base, CH)], idxv)
                pltpu.sync_copy(dx_hbm.at[pl.ds(base, CH)], rowv)
                pltpu.sync_copy(rowv, shared.at[idxv], add=True)

        plsc.subcore_barrier()
        pltpu.sync_copy(shared.at[pl.ds(r0, RPSP)],
                        out_hbm.at[cid, pl.ds(r0, RPSP)])

    return gather_kernel, scatter_pm_kernel, scatter_dx_kernel


def _gather_pair(feat, src, dst):
    return _sc_kernels()[0](feat, src, dst)


def _segment_pm(pm, dst, zeros_pm):
    return _sc_kernels()[1](pm, dst, zeros_pm)


def _segment_dx(dx, dst, zeros_dx):
    return _sc_kernels()[2](dx, dst, zeros_dx)


def _init_body(ia_ref, ic_ref, xp_ref, A_ref, C_ref, fb_ref, feat_ref):
    ia = ia_ref[0, 0, :]
    ic = ic_ref[0, 0, :]
    oa = (ia[:, None] == lax.broadcasted_iota(jnp.int32, (1, KA), 1)).astype(_f32)
    oc = (ic[:, None] == lax.broadcasted_iota(jnp.int32, (1, 8), 1)).astype(_f32)
    h0 = (jnp.dot(oa, A_ref[...], preferred_element_type=_f32)
          + jnp.dot(oc, C_ref[...], preferred_element_type=_f32)
          + fb_ref[...])
    feat_ref[:, :HP] = _pack_h(h0)
    feat_ref[:, HP:HP + XW] = xp_ref[...]
    feat_ref[:, HP + XW:] = jnp.zeros((feat_ref.shape[0], F - HP - XW), _f32)


def _init_feat(ia3, ic3, xp, A_tab, C_tab8, fuse_b_row):
    nb = N // BN
    return pl.pallas_call(
        _init_body,
        grid=(nb,),
        in_specs=[
            pl.BlockSpec((1, 1, BN), lambda i: (i, 0, 0)),
            pl.BlockSpec((1, 1, BN), lambda i: (i, 0, 0)),
            pl.BlockSpec((BN, XW), lambda i: (i, 0)),
            pl.BlockSpec((KA, D), lambda i: (0, 0)),
            pl.BlockSpec((8, D), lambda i: (0, 0)),
            pl.BlockSpec((1, D), lambda i: (0, 0)),
        ],
        out_specs=pl.BlockSpec((BN, F), lambda i: (i, 0)),
        out_shape=jax.ShapeDtypeStruct((N, F), _f32),
    )(ia3, ic3, xp, A_tab, C_tab8, fuse_b_row)


_CEN_STEP = 10.0 / (RK - 1)
_INV_W = 1.0 / (10.0 / RK + 1e-12)


def _silu(v):
    return v * jax.nn.sigmoid(v)


def _pack_h(h):
    lo = lax.bitcast_convert_type(h[:, :HP].astype(_bf16).astype(_f32),
                                  jnp.uint32)
    hi = lax.bitcast_convert_type(h[:, HP:].astype(_bf16).astype(_f32),
                                  jnp.uint32)
    w = (lo >> 16) | (hi & jnp.uint32(0xFFFF0000))
    return lax.bitcast_convert_type(w, _f32)


def _unpack_h(pk):
    w = lax.bitcast_convert_type(pk, jnp.uint32)
    lo = lax.bitcast_convert_type(w << 16, _f32)
    hi = lax.bitcast_convert_type(w & jnp.uint32(0xFFFF0000), _f32)
    return jnp.concatenate([lo, hi], axis=1)


def _unpack_h_bf(pk):
    w = lax.bitcast_convert_type(pk, jnp.uint32)
    lo = lax.bitcast_convert_type(w << 16, _f32).astype(_bf16)
    hi = lax.bitcast_convert_type(w & jnp.uint32(0xFFFF0000), _f32).astype(_bf16)
    return lo, hi


def _silu_bf(v):
    vb = v.astype(_bf16)
    return vb * jax.nn.sigmoid(vb)


def _edge_body(hs_ref, hd_ref, ib_ref, W1d_ref, W1s_ref, W1r_ref, Et_ref,
               b1_ref, W2m_ref, b2m_ref, W2p_ref, b2p_ref, Wx1_ref, bx1_ref,
               wx2_ref, bx2_ref, pm_ref, dx_ref):
    hs_lo, hs_hi = _unpack_h_bf(hs_ref[:, :HP])
    hd_lo, hd_hi = _unpack_h_bf(hd_ref[:, :HP])
    xs = hs_ref[:, HP:HP + XW]
    xd = hd_ref[:, HP:HP + XW]
    rij = xd - xs
    dsq = jnp.sum(rij * rij, axis=1, keepdims=True)
    dist = jnp.sqrt(dsq)
    cen = lax.broadcasted_iota(jnp.int32, (1, RK), 1).astype(_f32) * _CEN_STEP
    z = (dist - cen) * _INV_W
    rbf = jnp.exp(-0.5 * z * z)
    ib = ib_ref[0, 0, :]
    oh = (ib[:, None] == lax.broadcasted_iota(jnp.int32, (1, 8), 1)).astype(_bf16)

    pre = (jnp.dot(hd_lo, W1d_ref[:HP], preferred_element_type=_f32)
           + jnp.dot(hd_hi, W1d_ref[HP:], preferred_element_type=_f32)
           + jnp.dot(hs_lo, W1s_ref[:HP], preferred_element_type=_f32)
           + jnp.dot(hs_hi, W1s_ref[HP:], preferred_element_type=_f32)
           + jnp.dot(rbf.astype(_bf16), W1r_ref[...], preferred_element_type=_f32)
           + jnp.dot(oh, Et_ref[...], preferred_element_type=_f32))
    m = _silu(pre[:, :D])
    pp = _silu(pre[:, D:])
    m2 = _silu(jnp.dot(m.astype(_bf16), W2m_ref[...],
                       preferred_element_type=_f32) + b2m_ref[...])
    pm = _silu(jnp.dot(pp.astype(_bf16), W2p_ref[...],
                       preferred_element_type=_f32) + b2p_ref[...])
    g1 = _silu(jnp.dot(m2.astype(_bf16), Wx1_ref[...],
                       preferred_element_type=_f32) + bx1_ref[...])
    gate = jnp.sum(g1 * wx2_ref[...], axis=1, keepdims=True) + bx2_ref[...]
    pm_ref[...] = pm
    dx_ref[...] = rij * gate


def _edge_call(hs, hd, ib3, w):
    nb = EH // BE
    full = lambda a: pl.BlockSpec(a.shape, lambda i: (0,) * a.ndim)
    return pl.pallas_call(
        _edge_body,
        grid=(nb,),
        in_specs=[
            pl.BlockSpec((BE, F), lambda i: (i, 0)),
            pl.BlockSpec((BE, F), lambda i: (i, 0)),
            pl.BlockSpec((1, 1, BE), lambda i: (i, 0, 0)),
        ] + [full(a) for a in w],
        out_specs=[
            pl.BlockSpec((BE, D), lambda i: (i, 0)),
            pl.BlockSpec((BE, DXW), lambda i: (i, 0)),
        ],
        out_shape=[
            jax.ShapeDtypeStruct((EH, D), _f32),
            jax.ShapeDtypeStruct((EH, DXW), _f32),
        ],
    )(hs, hd, ib3, *w)


def _node_body(feat_ref, p0_ref, p1_ref, p2_ref, p3_ref,
               q0_ref, q1_ref, q2_ref, q3_ref, Wh_ref, Wp_ref,
               b1_ref, W2_ref, b2_ref, g_ref, bln_ref, nf_ref, hsum_ref):
    i = pl.program_id(0)
    h = _unpack_h(feat_ref[:, :HP])
    xsl = feat_ref[:, HP:HP + XW]
    pm = (p0_ref[0] + p1_ref[0]) + (p2_ref[0] + p3_ref[0])
    dx = (q0_ref[0] + q1_ref[0]) + (q2_ref[0] + q3_ref[0])
    u1 = _silu(jnp.dot(h.astype(_bf16), Wh_ref[...], preferred_element_type=_f32)
               + jnp.dot(pm.astype(_bf16), Wp_ref[...], preferred_element_type=_f32)
               + b1_ref[...])
    u = jnp.dot(u1.astype(_bf16), W2_ref[...],
                preferred_element_type=_f32) + b2_ref[...]
    hr = h + u
    mu = jnp.mean(hr, axis=1, keepdims=True)
    var = jnp.mean((hr - mu) ** 2, axis=1, keepdims=True)
    hn = (hr - mu) / jnp.sqrt(var + 1e-5) * g_ref[...] + bln_ref[...]
    nf_ref[:, :HP] = _pack_h(hn)
    nf_ref[:, HP:HP + XW] = xsl + dx
    nf_ref[:, HP + XW:] = jnp.zeros((nf_ref.shape[0], F - HP - XW), _f32)
    ps = jnp.sum(hn, axis=0, keepdims=True)

    @pl.when(i == 0)
    def _():
        hsum_ref[...] = ps

    @pl.when(i > 0)
    def _():
        hsum_ref[...] += ps


def _node_call(feat, pm_a, pm_b, dx_a, dx_b, w):
    nb = N // BN
    full = lambda a: pl.BlockSpec(a.shape, lambda i: (0,) * a.ndim)
    return pl.pallas_call(
        _node_body,
        grid=(nb,),
        in_specs=[
            pl.BlockSpec((BN, F), lambda i: (i, 0)),
            pl.BlockSpec((1, BN, D), lambda i: (0, i, 0)),
            pl.BlockSpec((1, BN, D), lambda i: (1, i, 0)),
            pl.BlockSpec((1, BN, D), lambda i: (0, i, 0)),
            pl.BlockSpec((1, BN, D), lambda i: (1, i, 0)),
            pl.BlockSpec((1, BN, DXW), lambda i: (0, i, 0)),
            pl.BlockSpec((1, BN, DXW), lambda i: (1, i, 0)),
            pl.BlockSpec((1, BN, DXW), lambda i: (0, i, 0)),
            pl.BlockSpec((1, BN, DXW), lambda i: (1, i, 0)),
        ] + [full(a) for a in w],
        out_specs=[
            pl.BlockSpec((BN, F), lambda i: (i, 0)),
            pl.BlockSpec((1, D), lambda i: (0, 0)),
        ],
        out_shape=[
            jax.ShapeDtypeStruct((N, F), _f32),
            jax.ShapeDtypeStruct((1, D), _f32),
        ],
    )(feat, pm_a, pm_a, pm_b, pm_b, dx_a, dx_a, dx_b, dx_b, *w)


def kernel(a_t, c_t, x_t, e_t, edge_index, W_atom, W_charge, W_bond, fuse_W,
           fuse_b, lift_W, lift_b, phim_W1, phim_b1, phim_W2, phim_b2,
           phix_W1, phix_b1, phix_W2, phix_b2, psim_W1, psim_b1, psim_W2,
           psim_b2, upd_W1, upd_b1, upd_W2, upd_b2, ln_g, ln_b, head_W,
           head_b):
    def amax(probs, key):
        logits = jnp.log(jnp.maximum(probs, 1e-12))
        g = jax.random.gumbel(key, logits.shape, logits.dtype)
        return jnp.argmax(logits + g, axis=-1).astype(jnp.int32)

    ia = amax(a_t, jax.random.key(11))
    ic = amax(c_t, jax.random.key(12))
    ib = amax(e_t, jax.random.key(13))
    ia3 = ia.reshape(N // BN, 1, BN)
    ic3 = ic.reshape(N // BN, 1, BN)
    ib3a = ib[:EH].reshape(EH // BE, 1, BE)
    ib3b = ib[EH:].reshape(EH // BE, 1, BE)

    A_tab = W_atom @ fuse_W[:DE]
    C_tab = W_charge @ fuse_W[DE:2 * DE]
    C_tab8 = jnp.zeros((8, D), _f32).at[:KC].set(C_tab)
    B_tab = W_bond @ lift_W
    fuse_b_row = fuse_b.reshape(1, D)

    xp = jnp.concatenate([x_t, jnp.zeros((N, XW - 3), _f32)], axis=1)
    feat = _init_feat(ia3, ic3, xp, A_tab, C_tab8, fuse_b_row)

    src_a = edge_index[0, :EH]
    src_b = edge_index[0, EH:]
    dst_a = edge_index[1, :EH]
    dst_b = edge_index[1, EH:]
    zeros_pm = jnp.zeros((NP, D), _f32)
    zeros_dx = jnp.zeros((NP, DXW), _f32)

    edge_w = []
    node_w = []
    for l in range(DEPTH):
        W1 = jnp.concatenate([phim_W1[l], psim_W1[l]], axis=1)
        b1v = (jnp.concatenate([phim_b1[l], psim_b1[l]])
               + lift_b @ W1[2 * D + RK:])
        Et = jnp.zeros((8, 2 * D), _f32).at[:KB].set(B_tab @ W1[2 * D + RK:]
                                                     + b1v)
        b1 = jnp.zeros((1, 2 * D), _f32)
        edge_w.append([
            W1[:D].astype(_bf16),
            W1[D:2 * D].astype(_bf16),
            W1[2 * D:2 * D + RK].astype(_bf16),
            Et.astype(_bf16),
            b1,
            phim_W2[l].astype(_bf16),
            phim_b2[l].reshape(1, D),
            psim_W2[l].astype(_bf16),
            psim_b2[l].reshape(1, D),
            phix_W1[l].astype(_bf16),
            phix_b1[l].reshape(1, D),
            phix_W2[l].reshape(1, D),
            phix_b2[l].reshape(1, 1),
        ])
        node_w.append([
            upd_W1[l][:D].astype(_bf16),
            upd_W1[l][D:].astype(_bf16),
            upd_b1[l].reshape(1, D),
            upd_W2[l].astype(_bf16),
            upd_b2[l].reshape(1, D),
            ln_g[l].reshape(1, D),
            ln_b[l].reshape(1, D),
        ])

    hsum = None
    for l in range(DEPTH):
        hs_a, hd_a = _gather_pair(feat, src_a, dst_a)
        hs_b, hd_b = _gather_pair(feat, src_b, dst_b)
        pm_a, dx_a = _edge_call(hs_a, hd_a, ib3a, edge_w[l])
        pm_b, dx_b = _edge_call(hs_b, hd_b, ib3b, edge_w[l])
        parts_pm_a = _segment_pm(pm_a, dst_a, zeros_pm)
        parts_pm_b = _segment_pm(pm_b, dst_b, zeros_pm)
        parts_dx_a = _segment_dx(dx_a, dst_a, zeros_dx)
        parts_dx_b = _segment_dx(dx_b, dst_b, zeros_dx)
        feat, hsum = _node_call(feat, parts_pm_a, parts_pm_b,
                                parts_dx_a, parts_dx_b, node_w[l])

    hg = hsum[0] / N
    out = hg @ head_W + head_b
    return jax.nn.sigmoid(out / 2.0)
